# baseline (speedup 1.0000x reference)
import jax
import jax.numpy as jnp
from jax import lax
from jax.experimental import pallas as pl
from jax.experimental.pallas import tpu as pltpu
from jax.experimental.pallas import tpu_sc as plsc

_P = 35
_T = 4
_CV = 2
_DV = 2
_B = 4096
_L = 50
_SMOOTH = 0.1

_NC = 2
_NS = 16
_NW = _NC * _NS
_SENT_PER_W = _B // _NW
_EV_PER_W = _SENT_PER_W * _L

_TRANS_BINS = _P * _P * _T * _T * 2 * _CV
_DEC_BINS = _P * _T * 2 * _DV * 2
_DEC_BASE = _TRANS_BINS
_HIST = _TRANS_BINS + _DEC_BINS
_ZSLICE = 4992
_HIST_PAD = _ZSLICE * 16

_ROWS_PER_GROUP = _EV_PER_W // 128
_N_ROWS = 4 * _ROWS_PER_GROUP


def _sc_hist_body(pk_hbm, w_hbm, hist_out, pk_vm, w_vm, ib, wb, zb, shared,
                  sem):
    cid = lax.axis_index("c")
    sid = lax.axis_index("s")
    wid = cid * _NS + sid
    s0 = wid * _SENT_PER_W

    pltpu.sync_copy(pk_hbm.at[pl.ds(s0, _SENT_PER_W)], pk_vm)
    pltpu.sync_copy(w_hbm.at[pl.ds(s0, _SENT_PER_W)], w_vm)

    fz = jnp.zeros((16,), jnp.float32)

    def _zero(i, _):
        zb[pl.ds(i * 16, 16)] = fz
        return 0
    lax.fori_loop(0, _ZSLICE // 16, _zero, 0)
    pltpu.sync_copy(zb, shared.at[pl.ds(sid * _ZSLICE, _ZSLICE)])
    plsc.subcore_barrier()

    lane = lax.iota(jnp.int32, 16)
    iz = jnp.zeros((16,), jnp.int32)
    io = iz + 1
    i3 = iz + 3
    i63 = iz + 63
    iL = iz + _L

    def _chunk_row(j, _):
        for k in range(8):
            ev = (j * 128 + k * 16) + lane
            s = lax.div(ev, iL)
            t = ev - s * _L

            g = plsc.load_gather(pk_vm, [s, t])
            h = lax.shift_right_logical(g, iz + 11)
            mp = g & i63
            mt = lax.shift_right_logical(g, iz + 6) & i3
            hv = lax.shift_right_logical(g, iz + 8) & io
            v0 = lax.shift_right_logical(g, iz + 9) & io
            v1 = lax.shift_right_logical(g, iz + 10) & io
            gh = plsc.load_gather(pk_vm, [s, h])
            hp = gh & i63
            ht = lax.shift_right_logical(gh, iz + 6) & i3
            w = plsc.load_gather(w_vm, [s, t])

            d = jnp.where(h < t, io, iz)
            wh = jnp.where(h > iz, w, fz)

            tidx = ((((hp * _P + mp) * _T + ht) * _T + mt) * 2 + d) * _CV + hv
            bm = (mp * _T + mt) * 8
            bh = (hp * _T + ht) * 8
            d1 = _DEC_BASE + bm + v0 * 2
            d2 = _DEC_BASE + bm + 4 + v1 * 2
            d3 = _DEC_BASE + bh + d * 4 + hv * 2 + 1

            c = k * 16
            ib[j, pl.ds(c, 16)] = tidx
            wb[j, pl.ds(c, 16)] = w
            ib[_ROWS_PER_GROUP + j, pl.ds(c, 16)] = d1
            wb[_ROWS_PER_GROUP + j, pl.ds(c, 16)] = w
            ib[2 * _ROWS_PER_GROUP + j, pl.ds(c, 16)] = d2
            wb[2 * _ROWS_PER_GROUP + j, pl.ds(c, 16)] = w
            ib[3 * _ROWS_PER_GROUP + j, pl.ds(c, 16)] = d3
            wb[3 * _ROWS_PER_GROUP + j, pl.ds(c, 16)] = wh
        return 0

    lax.fori_loop(0, _ROWS_PER_GROUP, _chunk_row, 0)

    for g in range(_N_ROWS // 8):
        descs = []
        for r in range(g * 8, g * 8 + 8):
            descs.append(
                pltpu.async_copy(wb.at[r], shared.at[ib.at[r]], sem, add=True))
        for dsc in descs:
            dsc.wait()
    plsc.subcore_barrier()

    off = sid * _ZSLICE
    pltpu.sync_copy(shared.at[pl.ds(off, _ZSLICE)], zb)
    pltpu.sync_copy(zb, hist_out.at[pl.ds(cid * _HIST_PAD + off, _ZSLICE)])


@jax.jit
def _sc_hist(pk, weights):
    mesh = plsc.VectorSubcoreMesh(core_axis_name="c", subcore_axis_name="s")
    f = pl.kernel(
        _sc_hist_body,
        out_type=jax.ShapeDtypeStruct((_NC * _HIST_PAD,), jnp.float32),
        mesh=mesh,
        scratch_types=[
            pltpu.VMEM((_SENT_PER_W, _L), jnp.int32),
            pltpu.VMEM((_SENT_PER_W, _L), jnp.float32),
            pltpu.VMEM((_N_ROWS, 128), jnp.int32),
            pltpu.VMEM((_N_ROWS, 128), jnp.float32),
            pltpu.VMEM((_ZSLICE,), jnp.float32),
            pltpu.VMEM_SHARED((_HIST_PAD,), jnp.float32),
            pltpu.SemaphoreType.DMA,
        ],
        compiler_params=pltpu.CompilerParams(needs_layout_passes=False),
    )
    return f(pk, weights)


def _finalize_body(tp_ref, dp_ref, to_ref, do_ref):
    t = tp_ref[0] + tp_ref[1] + _SMOOTH
    to_ref[...] = t / jnp.sum(t, axis=1, keepdims=True)
    d = dp_ref[0] + dp_ref[1] + _SMOOTH
    do_ref[...] = d / jnp.sum(d, axis=1, keepdims=True)


@jax.jit
def kernel(pos_ids, heads, tags, head_valences, valences, weights):
    pk = (pos_ids | (tags << 6) | (head_valences << 8)
          | (valences[..., 0] << 9) | (valences[..., 1] << 10)
          | (heads << 11))
    hist = _sc_hist(pk, weights).reshape(_NC, _HIST_PAD)
    tp = hist[:, :_TRANS_BINS].reshape(_NC, _P, _P, _T * _T * 2 * _CV)
    dp = hist[:, _DEC_BASE:_HIST].reshape(_NC, _P * _T * 2, _DV, 2)
    tparam, dparam = pl.pallas_call(
        _finalize_body,
        out_shape=(
            jax.ShapeDtypeStruct((_P, _P, _T * _T * 2 * _CV), jnp.float32),
            jax.ShapeDtypeStruct((_P * _T * 2, _DV, 2), jnp.float32),
        ),
    )(tp, dp)
    return jnp.concatenate([tparam.ravel(), dparam.ravel()])


# speedup vs baseline: 1.2600x; 1.2600x over previous
import jax
import jax.numpy as jnp
from jax import lax
from jax.experimental import pallas as pl
from jax.experimental.pallas import tpu as pltpu
from jax.experimental.pallas import tpu_sc as plsc

_P = 35
_T = 4
_CV = 2
_DV = 2
_B = 4096
_L = 50
_SMOOTH = 0.1

_NC = 2
_NS = 16
_NW = _NC * _NS
_SENT_PER_W = _B // _NW
_EV_PER_W = _SENT_PER_W * _L

_TRANS_BINS = _P * _P * _T * _T * 2 * _CV
_DEC_BINS = _P * _T * 2 * _DV * 2
_DEC_BASE = _TRANS_BINS
_HIST = _TRANS_BINS + _DEC_BINS
_ZSLICE = 4992
_HIST_PAD = _ZSLICE * 16

_ROWS_PER_GROUP = _EV_PER_W // 128
_N_ROWS = 4 * _ROWS_PER_GROUP


def _sc_hist_body(pk_hbm, w_hbm, hist_out, pk_vm, w_vm, ib, wb, zb, shared,
                  sem):
    cid = lax.axis_index("c")
    sid = lax.axis_index("s")
    wid = cid * _NS + sid
    s0 = wid * _SENT_PER_W

    ev0 = wid * _EV_PER_W
    pltpu.sync_copy(pk_hbm.at[pl.ds(ev0, _EV_PER_W)], pk_vm)
    pltpu.sync_copy(w_hbm.at[pl.ds(ev0, _EV_PER_W)], w_vm)

    fz = jnp.zeros((16,), jnp.float32)

    def _zero(i, _):
        zb[pl.ds(i * 16, 16)] = fz
        return 0
    lax.fori_loop(0, _ZSLICE // 16, _zero, 0)
    pltpu.sync_copy(zb, shared.at[pl.ds(sid * _ZSLICE, _ZSLICE)])
    plsc.subcore_barrier()

    lane = lax.iota(jnp.int32, 16)
    iz = jnp.zeros((16,), jnp.int32)
    io = iz + 1
    i3 = iz + 3
    i63 = iz + 63
    iL = iz + _L

    def _chunk_row(j, _):
        for k in range(8):
            off = j * 128 + k * 16
            ev = off + lane
            s = lax.div(ev, iL)
            t = ev - s * _L

            g = pk_vm[pl.ds(off, 16)]
            h = lax.shift_right_logical(g, iz + 11)
            mp = g & i63
            mt = lax.shift_right_logical(g, iz + 6) & i3
            hv = lax.shift_right_logical(g, iz + 8) & io
            v0 = lax.shift_right_logical(g, iz + 9) & io
            v1 = lax.shift_right_logical(g, iz + 10) & io
            gh = plsc.load_gather(pk_vm, [s * _L + h])
            hp = gh & i63
            ht = lax.shift_right_logical(gh, iz + 6) & i3
            w = w_vm[pl.ds(off, 16)]

            d = jnp.where(h < t, io, iz)
            wh = jnp.where(h > iz, w, fz)

            tidx = ((((hp * _P + mp) * _T + ht) * _T + mt) * 2 + d) * _CV + hv
            bm = (mp * _T + mt) * 8
            bh = (hp * _T + ht) * 8
            d1 = _DEC_BASE + bm + v0 * 2
            d2 = _DEC_BASE + bm + 4 + v1 * 2
            d3 = _DEC_BASE + bh + d * 4 + hv * 2 + 1

            c = k * 16
            ib[j, pl.ds(c, 16)] = tidx
            wb[j, pl.ds(c, 16)] = w
            ib[_ROWS_PER_GROUP + j, pl.ds(c, 16)] = d1
            wb[_ROWS_PER_GROUP + j, pl.ds(c, 16)] = w
            ib[2 * _ROWS_PER_GROUP + j, pl.ds(c, 16)] = d2
            wb[2 * _ROWS_PER_GROUP + j, pl.ds(c, 16)] = w
            ib[3 * _ROWS_PER_GROUP + j, pl.ds(c, 16)] = d3
            wb[3 * _ROWS_PER_GROUP + j, pl.ds(c, 16)] = wh
        return 0

    lax.fori_loop(0, _ROWS_PER_GROUP, _chunk_row, 0)

    for g in range(_N_ROWS // 8):
        descs = []
        for r in range(g * 8, g * 8 + 8):
            descs.append(
                pltpu.async_copy(wb.at[r], shared.at[ib.at[r]], sem, add=True))
        for dsc in descs:
            dsc.wait()
    plsc.subcore_barrier()

    off = sid * _ZSLICE
    pltpu.sync_copy(shared.at[pl.ds(off, _ZSLICE)], zb)
    pltpu.sync_copy(zb, hist_out.at[pl.ds(cid * _HIST_PAD + off, _ZSLICE)])


@jax.jit
def _sc_hist(pk, weights):
    mesh = plsc.VectorSubcoreMesh(core_axis_name="c", subcore_axis_name="s")
    f = pl.kernel(
        _sc_hist_body,
        out_type=jax.ShapeDtypeStruct((_NC * _HIST_PAD,), jnp.float32),
        mesh=mesh,
        scratch_types=[
            pltpu.VMEM((_EV_PER_W,), jnp.int32),
            pltpu.VMEM((_EV_PER_W,), jnp.float32),
            pltpu.VMEM((_N_ROWS, 128), jnp.int32),
            pltpu.VMEM((_N_ROWS, 128), jnp.float32),
            pltpu.VMEM((_ZSLICE,), jnp.float32),
            pltpu.VMEM_SHARED((_HIST_PAD,), jnp.float32),
            pltpu.SemaphoreType.DMA,
        ],
        compiler_params=pltpu.CompilerParams(needs_layout_passes=False),
    )
    return f(pk, weights)


def _finalize_body(tp_ref, dp_ref, to_ref, do_ref):
    t = tp_ref[0] + tp_ref[1] + _SMOOTH
    to_ref[...] = t / jnp.sum(t, axis=1, keepdims=True)
    d = dp_ref[0] + dp_ref[1] + _SMOOTH
    do_ref[...] = d / jnp.sum(d, axis=1, keepdims=True)


@jax.jit
def kernel(pos_ids, heads, tags, head_valences, valences, weights):
    pk = (pos_ids | (tags << 6) | (head_valences << 8)
          | (valences[..., 0] << 9) | (valences[..., 1] << 10)
          | (heads << 11))
    hist = _sc_hist(pk.reshape(-1), weights.reshape(-1)).reshape(_NC,
                                                                 _HIST_PAD)
    tp = hist[:, :_TRANS_BINS].reshape(_NC, _P, _P, _T * _T * 2 * _CV)
    dp = hist[:, _DEC_BASE:_HIST].reshape(_NC, _P * _T * 2, _DV, 2)
    tparam, dparam = pl.pallas_call(
        _finalize_body,
        out_shape=(
            jax.ShapeDtypeStruct((_P, _P, _T * _T * 2 * _CV), jnp.float32),
            jax.ShapeDtypeStruct((_P * _T * 2, _DV, 2), jnp.float32),
        ),
    )(tp, dp)
    return jnp.concatenate([tparam.ravel(), dparam.ravel()])


# speedup vs baseline: 1.2862x; 1.0208x over previous
import jax
import jax.numpy as jnp
from jax import lax
from jax.experimental import pallas as pl
from jax.experimental.pallas import tpu as pltpu
from jax.experimental.pallas import tpu_sc as plsc

_P = 35
_T = 4
_CV = 2
_DV = 2
_B = 4096
_L = 50
_SMOOTH = 0.1

_NC = 2
_NS = 16
_NW = _NC * _NS
_SENT_PER_W = _B // _NW
_EV_PER_W = _SENT_PER_W * _L

_TRANS_BINS = _P * _P * _T * _T * 2 * _CV
_DEC_BINS = _P * _T * 2 * _DV * 2
_DEC_BASE = _TRANS_BINS
_HIST = _TRANS_BINS + _DEC_BINS
_ZSLICE = 4992
_HIST_PAD = _ZSLICE * 16

_ROWS_PER_GROUP = _EV_PER_W // 128
_N_ROWS = 4 * _ROWS_PER_GROUP


def _sc_hist_body(pk_hbm, w_hbm, hist_out, pk_vm, w_vm, ib, wb, zb, shared,
                  sem):
    cid = lax.axis_index("c")
    sid = lax.axis_index("s")
    wid = cid * _NS + sid
    s0 = wid * _SENT_PER_W

    ev0 = wid * _EV_PER_W
    pltpu.sync_copy(pk_hbm.at[pl.ds(ev0, _EV_PER_W)], pk_vm)
    pltpu.sync_copy(w_hbm.at[pl.ds(ev0, _EV_PER_W)], w_vm)

    fz = jnp.zeros((16,), jnp.float32)

    def _zero(i, _):
        zb[pl.ds(i * 16, 16)] = fz
        return 0
    lax.fori_loop(0, _ZSLICE // 16, _zero, 0)
    pltpu.sync_copy(zb, shared.at[pl.ds(sid * _ZSLICE, _ZSLICE)])
    plsc.subcore_barrier()

    lane = lax.iota(jnp.int32, 16)
    iz = jnp.zeros((16,), jnp.int32)
    io = iz + 1
    i3 = iz + 3
    i63 = iz + 63
    iL = iz + _L

    def _chunk_row(j, carry):
        s, t = carry
        for k in range(8):
            off = j * 128 + k * 16

            g = pk_vm[pl.ds(off, 16)]
            h = lax.shift_right_logical(g, iz + 11)
            mp = g & i63
            mt = lax.shift_right_logical(g, iz + 6) & i3
            hv = lax.shift_right_logical(g, iz + 8) & io
            v0 = lax.shift_right_logical(g, iz + 9) & io
            v1 = lax.shift_right_logical(g, iz + 10) & io
            gh = plsc.load_gather(pk_vm, [s * _L + h])
            hp = gh & i63
            ht = lax.shift_right_logical(gh, iz + 6) & i3
            w = w_vm[pl.ds(off, 16)]

            d = jnp.where(h < t, io, iz)
            wh = jnp.where(h > iz, w, fz)

            tidx = ((((hp * _P + mp) * _T + ht) * _T + mt) * 2 + d) * _CV + hv
            bm = (mp * _T + mt) * 8
            bh = (hp * _T + ht) * 8
            d1 = _DEC_BASE + bm + v0 * 2
            d2 = _DEC_BASE + bm + 4 + v1 * 2
            d3 = _DEC_BASE + bh + d * 4 + hv * 2 + 1

            c = k * 16
            ib[j, pl.ds(c, 16)] = tidx
            wb[j, pl.ds(c, 16)] = w
            ib[_ROWS_PER_GROUP + j, pl.ds(c, 16)] = d1
            ib[2 * _ROWS_PER_GROUP + j, pl.ds(c, 16)] = d2
            ib[3 * _ROWS_PER_GROUP + j, pl.ds(c, 16)] = d3
            wb[_ROWS_PER_GROUP + j, pl.ds(c, 16)] = wh

            t16 = t + 16
            wrap = t16 >= iL
            t = jnp.where(wrap, t16 - _L, t16)
            s = jnp.where(wrap, s + 1, s)
        return s, t

    lax.fori_loop(0, _ROWS_PER_GROUP, _chunk_row, (iz, lane))

    for g in range(_N_ROWS // 8):
        descs = []
        for r in range(g * 8, g * 8 + 8):
            wrow = r if r < 50 else (r - 50 if r < 100 else r - 100)
            descs.append(
                pltpu.async_copy(wb.at[wrow], shared.at[ib.at[r]], sem,
                                 add=True))
        for dsc in descs:
            dsc.wait()
    plsc.subcore_barrier()

    off = sid * _ZSLICE
    pltpu.sync_copy(shared.at[pl.ds(off, _ZSLICE)], zb)
    pltpu.sync_copy(zb, hist_out.at[pl.ds(cid * _HIST_PAD + off, _ZSLICE)])


@jax.jit
def _sc_hist(pk, weights):
    mesh = plsc.VectorSubcoreMesh(core_axis_name="c", subcore_axis_name="s")
    f = pl.kernel(
        _sc_hist_body,
        out_type=jax.ShapeDtypeStruct((_NC * _HIST_PAD,), jnp.float32),
        mesh=mesh,
        scratch_types=[
            pltpu.VMEM((_EV_PER_W,), jnp.int32),
            pltpu.VMEM((_EV_PER_W,), jnp.float32),
            pltpu.VMEM((_N_ROWS, 128), jnp.int32),
            pltpu.VMEM((_N_ROWS // 2, 128), jnp.float32),
            pltpu.VMEM((_ZSLICE,), jnp.float32),
            pltpu.VMEM_SHARED((_HIST_PAD,), jnp.float32),
            pltpu.SemaphoreType.DMA,
        ],
        compiler_params=pltpu.CompilerParams(needs_layout_passes=False),
    )
    return f(pk, weights)


def _finalize_body(tp_ref, dp_ref, to_ref, do_ref):
    t = tp_ref[0] + tp_ref[1] + _SMOOTH
    to_ref[...] = t / jnp.sum(t, axis=1, keepdims=True)
    d = dp_ref[0] + dp_ref[1] + _SMOOTH
    do_ref[...] = d / jnp.sum(d, axis=1, keepdims=True)


@jax.jit
def kernel(pos_ids, heads, tags, head_valences, valences, weights):
    pk = (pos_ids | (tags << 6) | (head_valences << 8)
          | (valences[..., 0] << 9) | (valences[..., 1] << 10)
          | (heads << 11))
    hist = _sc_hist(pk.reshape(-1), weights.reshape(-1)).reshape(_NC,
                                                                 _HIST_PAD)
    tp = hist[:, :_TRANS_BINS].reshape(_NC, _P, _P, _T * _T * 2 * _CV)
    dp = hist[:, _DEC_BASE:_HIST].reshape(_NC, _P * _T * 2, _DV, 2)
    tparam, dparam = pl.pallas_call(
        _finalize_body,
        out_shape=(
            jax.ShapeDtypeStruct((_P, _P, _T * _T * 2 * _CV), jnp.float32),
            jax.ShapeDtypeStruct((_P * _T * 2, _DV, 2), jnp.float32),
        ),
    )(tp, dp)
    return jnp.concatenate([tparam.ravel(), dparam.ravel()])


# speedup vs baseline: 1.3048x; 1.0145x over previous
import jax
import jax.numpy as jnp
from jax import lax
from jax.experimental import pallas as pl
from jax.experimental.pallas import tpu as pltpu
from jax.experimental.pallas import tpu_sc as plsc

_P = 35
_T = 4
_CV = 2
_DV = 2
_B = 4096
_L = 50
_SMOOTH = 0.1

_NC = 2
_NS = 16
_NW = _NC * _NS
_SENT_PER_W = _B // _NW
_EV_PER_W = _SENT_PER_W * _L

_TRANS_BINS = _P * _P * _T * _T * 2 * _CV
_DEC_BINS = _P * _T * 2 * _DV * 2
_DEC_BASE = _TRANS_BINS
_HIST = _TRANS_BINS + _DEC_BINS
_ZSLICE = 4992
_HIST_PAD = _ZSLICE * 16

_ROWS_PER_GROUP = _EV_PER_W // 128
_DEC_PAD = 1152
_DEC_ROWS = _DEC_PAD // 128
_N_ROWS = _ROWS_PER_GROUP + _DEC_ROWS


def _sc_hist_body(pk_hbm, w_hbm, hist_out, pk_vm, w_vm, ib, wb, dec_vm, zb,
                  shared, sem):
    cid = lax.axis_index("c")
    sid = lax.axis_index("s")
    wid = cid * _NS + sid
    s0 = wid * _SENT_PER_W

    ev0 = wid * _EV_PER_W
    pltpu.sync_copy(pk_hbm.at[pl.ds(ev0, _EV_PER_W)], pk_vm)
    pltpu.sync_copy(w_hbm.at[pl.ds(ev0, _EV_PER_W)], w_vm)

    fz = jnp.zeros((16,), jnp.float32)

    def _zero(i, _):
        zb[pl.ds(i * 16, 16)] = fz
        return 0
    lax.fori_loop(0, _ZSLICE // 16, _zero, 0)
    pltpu.sync_copy(zb, shared.at[pl.ds(sid * _ZSLICE, _ZSLICE)])

    def _zdec(i, _):
        dec_vm[pl.ds(i * 16, 16)] = fz
        return 0
    lax.fori_loop(0, _DEC_PAD // 16, _zdec, 0)
    plsc.subcore_barrier()

    lane = lax.iota(jnp.int32, 16)
    iz = jnp.zeros((16,), jnp.int32)
    io = iz + 1
    i3 = iz + 3
    i63 = iz + 63
    iL = iz + _L

    def _chunk_row(j, carry):
        s, t = carry
        for k in range(8):
            off = j * 128 + k * 16

            g = pk_vm[pl.ds(off, 16)]
            h = lax.shift_right_logical(g, iz + 11)
            mp = g & i63
            mt = lax.shift_right_logical(g, iz + 6) & i3
            hv = lax.shift_right_logical(g, iz + 8) & io
            v0 = lax.shift_right_logical(g, iz + 9) & io
            v1 = lax.shift_right_logical(g, iz + 10) & io
            gh = plsc.load_gather(pk_vm, [s * _L + h])
            hp = gh & i63
            ht = lax.shift_right_logical(gh, iz + 6) & i3
            w = w_vm[pl.ds(off, 16)]

            d = jnp.where(h < t, io, iz)
            wh = jnp.where(h > iz, w, fz)

            tidx = ((((hp * _P + mp) * _T + ht) * _T + mt) * 2 + d) * _CV + hv
            bm = (mp * _T + mt) * 8
            bh = (hp * _T + ht) * 8

            c = k * 16
            ib[j, pl.ds(c, 16)] = tidx
            wb[j, pl.ds(c, 16)] = w
            plsc.addupdate_scatter(dec_vm, [bm + v0 * 2], w)
            plsc.addupdate_scatter(dec_vm, [bm + 4 + v1 * 2], w)
            plsc.addupdate_scatter(dec_vm, [bh + d * 4 + hv * 2 + 1], wh)

            t16 = t + 16
            wrap = t16 >= iL
            t = jnp.where(wrap, t16 - _L, t16)
            s = jnp.where(wrap, s + 1, s)
        return s, t

    lax.fori_loop(0, _ROWS_PER_GROUP, _chunk_row, (iz, lane))

    for i in range(_DEC_PAD // 16):
        ib[_ROWS_PER_GROUP + i // 8, pl.ds((i % 8) * 16, 16)] = (
            lane + (_DEC_BASE + i * 16))

    def _src(r):
        if r < _ROWS_PER_GROUP:
            return wb.at[r]
        return dec_vm.at[pl.ds((r - _ROWS_PER_GROUP) * 128, 128)]

    for g in range((_N_ROWS + 7) // 8):
        descs = []
        for r in range(g * 8, min(g * 8 + 8, _N_ROWS)):
            descs.append(
                pltpu.async_copy(_src(r), shared.at[ib.at[r]], sem, add=True))
        for dsc in descs:
            dsc.wait()
    plsc.subcore_barrier()

    off = sid * _ZSLICE
    pltpu.sync_copy(shared.at[pl.ds(off, _ZSLICE)], zb)
    pltpu.sync_copy(zb, hist_out.at[pl.ds(cid * _HIST_PAD + off, _ZSLICE)])


@jax.jit
def _sc_hist(pk, weights):
    mesh = plsc.VectorSubcoreMesh(core_axis_name="c", subcore_axis_name="s")
    f = pl.kernel(
        _sc_hist_body,
        out_type=jax.ShapeDtypeStruct((_NC * _HIST_PAD,), jnp.float32),
        mesh=mesh,
        scratch_types=[
            pltpu.VMEM((_EV_PER_W,), jnp.int32),
            pltpu.VMEM((_EV_PER_W,), jnp.float32),
            pltpu.VMEM((_N_ROWS, 128), jnp.int32),
            pltpu.VMEM((_ROWS_PER_GROUP, 128), jnp.float32),
            pltpu.VMEM((_DEC_PAD,), jnp.float32),
            pltpu.VMEM((_ZSLICE,), jnp.float32),
            pltpu.VMEM_SHARED((_HIST_PAD,), jnp.float32),
            pltpu.SemaphoreType.DMA,
        ],
        compiler_params=pltpu.CompilerParams(needs_layout_passes=False),
    )
    return f(pk, weights)


def _finalize_body(tp_ref, dp_ref, to_ref, do_ref):
    t = tp_ref[0] + tp_ref[1] + _SMOOTH
    to_ref[...] = t / jnp.sum(t, axis=1, keepdims=True)
    d = dp_ref[0] + dp_ref[1] + _SMOOTH
    do_ref[...] = d / jnp.sum(d, axis=1, keepdims=True)


@jax.jit
def kernel(pos_ids, heads, tags, head_valences, valences, weights):
    pk = (pos_ids | (tags << 6) | (head_valences << 8)
          | (valences[..., 0] << 9) | (valences[..., 1] << 10)
          | (heads << 11))
    hist = _sc_hist(pk.reshape(-1), weights.reshape(-1)).reshape(_NC,
                                                                 _HIST_PAD)
    tp = hist[:, :_TRANS_BINS].reshape(_NC, _P, _P, _T * _T * 2 * _CV)
    dp = hist[:, _DEC_BASE:_HIST].reshape(_NC, _P * _T * 2, _DV, 2)
    tparam, dparam = pl.pallas_call(
        _finalize_body,
        out_shape=(
            jax.ShapeDtypeStruct((_P, _P, _T * _T * 2 * _CV), jnp.float32),
            jax.ShapeDtypeStruct((_P * _T * 2, _DV, 2), jnp.float32),
        ),
    )(tp, dp)
    return jnp.concatenate([tparam.ravel(), dparam.ravel()])
